# Initial kernel scaffold; baseline (speedup 1.0000x reference)
import jax
import jax.numpy as jnp
from jax.experimental import pallas as pl


def kernel(x, edge_index, edge_attr, W1, b1, W2a, b2a, W2b, b2b):
    raise NotImplementedError("write your pallas kernel here")


# speedup vs baseline: 6.3738x; 6.3738x over previous
import functools

import jax
import jax.numpy as jnp
from jax import lax
from jax.experimental import pallas as pl
from jax.experimental.pallas import tpu as pltpu
from jax.experimental.pallas import tpu_sc as plsc

N = 50000
E = 1600000
FX = 16
FM = 4
NC, NS, L = 2, 16, 16
NW = NC * NS
NCHUNK = NW // FM
EPT = E // NCHUNK
CB = 2000
NSTEP = EPT // CB


def _sc_aggregate_body(xaT, src, dst, ecT, part, xa_col, agg_col,
                       s_buf0, s_buf1, d_buf0, d_buf1, e_buf0, e_buf1, sems):
  s_bufs = (s_buf0, s_buf1)
  d_bufs = (d_buf0, d_buf1)
  e_bufs = (e_buf0, e_buf1)
  wid = lax.axis_index("s") * NC + lax.axis_index("c")
  f = wid // NCHUNK
  ch = wid % NCHUNK
  base = ch * EPT

  pltpu.sync_copy(xaT.at[pl.ds(f * N, N)], xa_col)

  zero = jnp.zeros((L,), jnp.float32)

  @pl.loop(0, N // L)
  def _(i):
    agg_col[pl.ds(i * L, L)] = zero

  def chunk_copies(j, b):
    off = base + j * CB
    return (
        pltpu.make_async_copy(src.at[pl.ds(off, CB)], s_bufs[b], sems.at[b]),
        pltpu.make_async_copy(dst.at[pl.ds(off, CB)], d_bufs[b], sems.at[b]),
        pltpu.make_async_copy(ecT.at[pl.ds(f * E + off, CB)], e_bufs[b],
                              sems.at[b]),
    )

  def start(j, b):
    for c in chunk_copies(j, b):
      c.start()

  def wait(j, b):
    for c in chunk_copies(j, b):
      c.wait()

  def process(b):
    sb = s_bufs[b]
    db = d_bufs[b]
    eb = e_bufs[b]

    @pl.loop(0, CB // L)
    def _(i):
      o = i * L
      si = sb[pl.ds(o, L)]
      di = db[pl.ds(o, L)]
      v = plsc.load_gather(xa_col, [si]) + eb[pl.ds(o, L)]
      cur = plsc.load_gather(agg_col, [di])
      m0 = v > cur

      def cond(m):
        return jnp.any(m)

      def body(m):
        plsc.store_scatter(agg_col, [di], v, mask=m)
        cur2 = plsc.load_gather(agg_col, [di])
        return m & (v > cur2)

      lax.while_loop(cond, body, m0)

  start(0, 0)

  @pl.loop(0, NSTEP, step=2)
  def _(jj):
    for b in (0, 1):
      j = jj + b

      @pl.when(j + 1 < NSTEP)
      def _():
        start(j + 1, 1 - b)

      wait(j, b)
      process(b)

  pltpu.sync_copy(agg_col, part.at[pl.ds(wid * N, N)])


_sc_aggregate = functools.partial(
    pl.kernel,
    out_type=jax.ShapeDtypeStruct((NW * N,), jnp.float32),
    mesh=plsc.VectorSubcoreMesh(core_axis_name="c", subcore_axis_name="s"),
    compiler_params=pltpu.CompilerParams(needs_layout_passes=False),
    scratch_types=[
        pltpu.VMEM((N,), jnp.float32),
        pltpu.VMEM((N,), jnp.float32),
        pltpu.VMEM((CB,), jnp.int32),
        pltpu.VMEM((CB,), jnp.int32),
        pltpu.VMEM((CB,), jnp.int32),
        pltpu.VMEM((CB,), jnp.int32),
        pltpu.VMEM((CB,), jnp.float32),
        pltpu.VMEM((CB,), jnp.float32),
        pltpu.SemaphoreType.DMA((2,)),
    ],
)(_sc_aggregate_body)


_BR = 1000
_BN = 1024


def _ec_body(ea_ref, w_ref, b_ref, out_ref):
  out_ref[...] = ea_ref[...] @ w_ref[...] + b_ref[...]


_ec_call = pl.pallas_call(
    _ec_body,
    grid=(E // 16 // _BR,),
    in_specs=[
        pl.BlockSpec((_BR, 128), lambda i: (i, 0)),
        pl.BlockSpec((128, 64), lambda i: (0, 0)),
        pl.BlockSpec((1, 64), lambda i: (0, 0)),
    ],
    out_specs=pl.BlockSpec((_BR, 64), lambda i: (i, 0)),
    out_shape=jax.ShapeDtypeStruct((E // 16, 64), jnp.float32),
)


def _xa_body(xT_ref, w_ref, out_ref):
  out_ref[...] = w_ref[...] @ xT_ref[...]


_xa_call = pl.pallas_call(
    _xa_body,
    grid=(pl.cdiv(N, _BN),),
    in_specs=[
        pl.BlockSpec((FX, _BN), lambda i: (0, i)),
        pl.BlockSpec((FM, FX), lambda i: (0, 0)),
    ],
    out_specs=pl.BlockSpec((FM, _BN), lambda i: (0, i)),
    out_shape=jax.ShapeDtypeStruct((FM, N), jnp.float32),
)


def _update_body(part_ref, xT_ref, w2ax_ref, w2aa_ref, b2a_ref, w2b_ref,
                 b2b_ref, w1c_ref, w1x_ref, xn_ref, xan_ref):
  p = part_ref[...]
  aggr = p[:, 0, :]
  for k in range(1, NCHUNK):
    aggr = jnp.maximum(aggr, p[:, k, :])
  xT = xT_ref[...]
  h = w2ax_ref[...] @ xT + w2aa_ref[...] @ aggr + b2a_ref[...]
  h = jnp.maximum(h, 0.0)
  comb = w2b_ref[...] @ h + b2b_ref[...]
  nor = jnp.sqrt(jnp.sum(comb * comb, axis=0, keepdims=True))
  comb = comb / jnp.maximum(1.0, nor)
  x8 = xT[0:8, :]
  xn_ref[0:8, :] = comb
  xn_ref[8:16, :] = x8
  xan_ref[...] = w1c_ref[...] @ comb + w1x_ref[...] @ x8


_update_call = pl.pallas_call(
    _update_body,
    grid=(pl.cdiv(N, _BN),),
    in_specs=[
        pl.BlockSpec((FM, NCHUNK, _BN), lambda i: (0, 0, i)),
        pl.BlockSpec((FX, _BN), lambda i: (0, i)),
        pl.BlockSpec((FM, FX), lambda i: (0, 0)),
        pl.BlockSpec((FM, FM), lambda i: (0, 0)),
        pl.BlockSpec((FM, 1), lambda i: (0, 0)),
        pl.BlockSpec((8, FM), lambda i: (0, 0)),
        pl.BlockSpec((8, 1), lambda i: (0, 0)),
        pl.BlockSpec((FM, 8), lambda i: (0, 0)),
        pl.BlockSpec((FM, 8), lambda i: (0, 0)),
    ],
    out_specs=[
        pl.BlockSpec((FX, _BN), lambda i: (0, i)),
        pl.BlockSpec((FM, _BN), lambda i: (0, i)),
    ],
    out_shape=[
        jax.ShapeDtypeStruct((FX, N), jnp.float32),
        jax.ShapeDtypeStruct((FM, N), jnp.float32),
    ],
)


def kernel(x, edge_index, edge_attr, W1, b1, W2a, b2a, W2b, b2b):
  src = edge_index[0]
  dst = edge_index[1]

  w_big = jnp.kron(jnp.eye(16, dtype=jnp.float32), W1[FX:])
  b_big = jnp.tile(b1, 16).reshape(1, 64)
  ec = _ec_call(edge_attr.reshape(E // 16, 128), w_big, b_big)
  ecT = jnp.transpose(ec.reshape(E, FM)).reshape(FM * E)

  xT = jnp.transpose(x)
  w1xT = jnp.transpose(W1[:FX])
  xaT = _xa_call(xT, w1xT)

  w2ax = jnp.transpose(W2a[:FX])
  w2aa = jnp.transpose(W2a[FX:])
  w2b = jnp.transpose(W2b)
  b2a_c = b2a.reshape(FM, 1)
  b2b_c = b2b.reshape(8, 1)
  w1c = w1xT[:, 0:8]
  w1x = w1xT[:, 8:16]

  for _ in range(3):
    part = _sc_aggregate(xaT.reshape(FM * N), src, dst, ecT)
    xT, xaT = _update_call(part.reshape(FM, NCHUNK, N), xT, w2ax, w2aa,
                           b2a_c, w2b, b2b_c, w1c, w1x)

  return jnp.transpose(xT)


# speedup vs baseline: 11.2608x; 1.7667x over previous
import functools

import jax
import jax.numpy as jnp
from jax import lax
from jax.experimental import pallas as pl
from jax.experimental.pallas import tpu as pltpu
from jax.experimental.pallas import tpu_sc as plsc

N = 50000
E = 1600000
FX = 16
FM = 4
NC, NS, L = 2, 16, 16
NW = NC * NS
NCHUNK = NW // FM
EPT = E // NCHUNK
CB = 2000
NSTEP = EPT // CB


def _mm(a, b):
  return jnp.matmul(a, b, precision=lax.Precision.HIGHEST)


def _mmt(a, b):
  return lax.dot_general(a, b, (((0,), (0,)), ((), ())),
                         preferred_element_type=jnp.float32,
                         precision=lax.Precision.HIGHEST)


def _sc_aggregate_body(xaT, sd, ecT, part, xa_col, agg_col,
                       s_buf0, s_buf1, e_buf0, e_buf1, sems):
  s_bufs = (s_buf0, s_buf1)
  e_bufs = (e_buf0, e_buf1)
  wid = lax.axis_index("s") * NC + lax.axis_index("c")
  f = wid // NCHUNK
  ch = wid % NCHUNK
  base = ch * EPT

  pltpu.sync_copy(xaT.at[pl.ds(f * N, N)], xa_col)

  zero = jnp.zeros((L,), jnp.float32)

  @pl.loop(0, N // L)
  def _(i):
    agg_col[pl.ds(i * L, L)] = zero

  def chunk_copies(j, b):
    off = base + j * CB
    return (
        pltpu.make_async_copy(sd.at[pl.ds(off, CB)], s_bufs[b], sems.at[b]),
        pltpu.make_async_copy(ecT.at[pl.ds(f * E + off, CB)], e_bufs[b],
                              sems.at[b]),
    )

  def start(j, b):
    for c in chunk_copies(j, b):
      c.start()

  def wait(j, b):
    for c in chunk_copies(j, b):
      c.wait()

  def process(b):
    sb = s_bufs[b]
    eb = e_bufs[b]

    def one_pass(_):
      def group(i, viol):
        o = i * L
        sdv = sb[pl.ds(o, L)]
        si = sdv & 0xFFFF
        di = lax.shift_right_logical(sdv, 16)
        v = plsc.load_gather(xa_col, [si]) + eb[pl.ds(o, L)]
        cur = plsc.load_gather(agg_col, [di])
        m = v > cur
        plsc.store_scatter(agg_col, [di], v, mask=m)
        cur2 = plsc.load_gather(agg_col, [di])
        return viol | (m & (v > cur2))

      viol = lax.fori_loop(0, CB // L, group,
                           jnp.zeros((L,), jnp.bool_), unroll=4)
      return jnp.any(viol)

    lax.while_loop(lambda go: go, one_pass, jnp.bool_(True))

  start(0, 0)

  @pl.loop(0, NSTEP, step=2)
  def _(jj):
    for b in (0, 1):
      j = jj + b

      @pl.when(j + 1 < NSTEP)
      def _():
        start(j + 1, 1 - b)

      wait(j, b)
      process(b)

  pltpu.sync_copy(agg_col, part.at[pl.ds(wid * N, N)])


_sc_aggregate = functools.partial(
    pl.kernel,
    out_type=jax.ShapeDtypeStruct((NW * N,), jnp.float32),
    mesh=plsc.VectorSubcoreMesh(core_axis_name="c", subcore_axis_name="s"),
    compiler_params=pltpu.CompilerParams(needs_layout_passes=False),
    scratch_types=[
        pltpu.VMEM((N,), jnp.float32),
        pltpu.VMEM((N,), jnp.float32),
        pltpu.VMEM((CB,), jnp.int32),
        pltpu.VMEM((CB,), jnp.int32),
        pltpu.VMEM((CB,), jnp.float32),
        pltpu.VMEM((CB,), jnp.float32),
        pltpu.SemaphoreType.DMA((2,)),
    ],
)(_sc_aggregate_body)


_BR = 1000
_BE = _BR * 16
_GE = E // _BE
_BN = 1024


def _edge_prep_body(ea_ref, w_ref, b_ref, ec_ref):
  ea = ea_ref[...]
  for f in range(FM):
    ec_ref[f] = _mm(ea, w_ref[f]) + b_ref[f]


_edge_prep = pl.pallas_call(
    _edge_prep_body,
    grid=(_GE,),
    in_specs=[
        pl.BlockSpec((_BR, 128), lambda i: (i, 0)),
        pl.BlockSpec((FM, 128, 16), lambda i: (0, 0, 0)),
        pl.BlockSpec((FM, 1), lambda i: (0, 0)),
    ],
    out_specs=pl.BlockSpec((FM, _BR, 16), lambda i: (0, i, 0)),
    out_shape=jax.ShapeDtypeStruct((FM, E // 16, 16), jnp.float32),
)


def _pack_body(sr_ref, dr_ref, sd_ref):
  sd_ref[...] = sr_ref[...] | lax.shift_left(dr_ref[...], 16)


_pack_call = pl.pallas_call(
    _pack_body,
    grid=(1,),
    in_specs=[
        pl.BlockSpec((50, E // 50), lambda i: (0, 0)),
        pl.BlockSpec((50, E // 50), lambda i: (0, 0)),
    ],
    out_specs=pl.BlockSpec((50, E // 50), lambda i: (0, 0)),
    out_shape=jax.ShapeDtypeStruct((50, E // 50), jnp.int32),
)


def _node_prep_body(x_ref, i16_ref, w_ref, xT_ref, xaT_ref):
  xTb = lax.dot_general(i16_ref[...], x_ref[...], (((1,), (1,)), ((), ())),
                        preferred_element_type=jnp.float32,
                        precision=lax.Precision.HIGHEST)
  xT_ref[...] = xTb
  xaT_ref[...] = _mm(w_ref[...], xTb)


_node_prep = pl.pallas_call(
    _node_prep_body,
    grid=(pl.cdiv(N, _BN),),
    in_specs=[
        pl.BlockSpec((_BN, FX), lambda i: (i, 0)),
        pl.BlockSpec((FX, FX), lambda i: (0, 0)),
        pl.BlockSpec((FM, FX), lambda i: (0, 0)),
    ],
    out_specs=[
        pl.BlockSpec((FX, _BN), lambda i: (0, i)),
        pl.BlockSpec((FM, _BN), lambda i: (0, i)),
    ],
    out_shape=[
        jax.ShapeDtypeStruct((FX, N), jnp.float32),
        jax.ShapeDtypeStruct((FM, N), jnp.float32),
    ],
)


def _combine(part_ref, xT_ref, w2ax_ref, w2aa_ref, b2a_ref, w2b_ref, b2b_ref):
  p = part_ref[...]
  aggr = p[:, 0, :]
  for k in range(1, NCHUNK):
    aggr = jnp.maximum(aggr, p[:, k, :])
  xT = xT_ref[...]
  h = _mm(w2ax_ref[...], xT) + _mm(w2aa_ref[...], aggr) + b2a_ref[...]
  h = jnp.maximum(h, 0.0)
  comb = _mm(w2b_ref[...], h) + b2b_ref[...]
  nor = jnp.sqrt(jnp.sum(comb * comb, axis=0, keepdims=True))
  comb = comb / jnp.maximum(1.0, nor)
  return comb, xT[0:8, :]


def _update_body(part_ref, xT_ref, w2ax_ref, w2aa_ref, b2a_ref, w2b_ref,
                 b2b_ref, w1c_ref, w1x_ref, xn_ref, xan_ref):
  comb, x8 = _combine(part_ref, xT_ref, w2ax_ref, w2aa_ref, b2a_ref,
                      w2b_ref, b2b_ref)
  xn_ref[0:8, :] = comb
  xn_ref[8:16, :] = x8
  xan_ref[...] = _mm(w1c_ref[...], comb) + _mm(w1x_ref[...], x8)


_mid_specs = [
    pl.BlockSpec((FM, NCHUNK, _BN), lambda i: (0, 0, i)),
    pl.BlockSpec((FX, _BN), lambda i: (0, i)),
    pl.BlockSpec((FM, FX), lambda i: (0, 0)),
    pl.BlockSpec((FM, FM), lambda i: (0, 0)),
    pl.BlockSpec((FM, 1), lambda i: (0, 0)),
    pl.BlockSpec((8, FM), lambda i: (0, 0)),
    pl.BlockSpec((8, 1), lambda i: (0, 0)),
]

_update_call = pl.pallas_call(
    _update_body,
    grid=(pl.cdiv(N, _BN),),
    in_specs=_mid_specs + [
        pl.BlockSpec((FM, 8), lambda i: (0, 0)),
        pl.BlockSpec((FM, 8), lambda i: (0, 0)),
    ],
    out_specs=[
        pl.BlockSpec((FX, _BN), lambda i: (0, i)),
        pl.BlockSpec((FM, _BN), lambda i: (0, i)),
    ],
    out_shape=[
        jax.ShapeDtypeStruct((FX, N), jnp.float32),
        jax.ShapeDtypeStruct((FM, N), jnp.float32),
    ],
)


def _final_body(part_ref, xT_ref, w2ax_ref, w2aa_ref, b2a_ref, w2b_ref,
                b2b_ref, ea_ref, eb_ref, out_ref):
  comb, x8 = _combine(part_ref, xT_ref, w2ax_ref, w2aa_ref, b2a_ref,
                      w2b_ref, b2b_ref)
  out_ref[...] = _mmt(comb, ea_ref[...]) + _mmt(x8, eb_ref[...])


_final_call = pl.pallas_call(
    _final_body,
    grid=(pl.cdiv(N, _BN),),
    in_specs=_mid_specs + [
        pl.BlockSpec((8, FX), lambda i: (0, 0)),
        pl.BlockSpec((8, FX), lambda i: (0, 0)),
    ],
    out_specs=pl.BlockSpec((_BN, FX), lambda i: (i, 0)),
    out_shape=jax.ShapeDtypeStruct((N, FX), jnp.float32),
)


def kernel(x, edge_index, edge_attr, W1, b1, W2a, b2a, W2b, b2b):
  src = edge_index[0].reshape(50, E // 50)
  dst = edge_index[1].reshape(50, E // 50)

  w4 = jnp.einsum("kf,aA->fakA", W1[FX:], jnp.eye(16, dtype=jnp.float32))
  w4 = w4.reshape(FM, 128, 16)
  ec4 = _edge_prep(edge_attr.reshape(E // 16, 128), w4, b1.reshape(FM, 1))
  ecT = ec4.reshape(FM * E)
  sd = _pack_call(src, dst).reshape(E)

  i16 = jnp.eye(FX, dtype=jnp.float32)
  w1xT = jnp.transpose(W1[:FX])
  xT, xaT = _node_prep(x, i16, w1xT)

  w2ax = jnp.transpose(W2a[:FX])
  w2aa = jnp.transpose(W2a[FX:])
  w2b = jnp.transpose(W2b)
  b2a_c = b2a.reshape(FM, 1)
  b2b_c = b2b.reshape(8, 1)
  w1c = w1xT[:, 0:8]
  w1x = w1xT[:, 8:16]
  e_hi = jnp.eye(8, FX, dtype=jnp.float32)
  e_lo = jnp.eye(8, FX, k=8, dtype=jnp.float32)

  for _ in range(2):
    part = _sc_aggregate(xaT.reshape(FM * N), sd, ecT)
    xT, xaT = _update_call(part.reshape(FM, NCHUNK, N), xT, w2ax, w2aa,
                           b2a_c, w2b, b2b_c, w1c, w1x)

  part = _sc_aggregate(xaT.reshape(FM * N), sd, ecT)
  return _final_call(part.reshape(FM, NCHUNK, N), xT, w2ax, w2aa,
                     b2a_c, w2b, b2b_c, e_hi, e_lo)


# speedup vs baseline: 12.3990x; 1.1011x over previous
import functools

import jax
import jax.numpy as jnp
from jax import lax
from jax.experimental import pallas as pl
from jax.experimental.pallas import tpu as pltpu
from jax.experimental.pallas import tpu_sc as plsc

N = 50000
E = 1600000
FX = 16
FM = 4
NC, NS, L = 2, 16, 16
NW = NC * NS
NCHUNK = NW // FM
EPT = E // NCHUNK
CB = 2000
NSTEP = EPT // CB


def _mm(a, b):
  return jnp.matmul(a, b)


def _mmt(a, b):
  return lax.dot_general(a, b, (((0,), (0,)), ((), ())),
                         preferred_element_type=jnp.float32,
                         precision=lax.Precision.HIGHEST)


def _sc_aggregate_body(xaT, sd, ecT, part, xa_col, agg_col,
                       s_buf0, s_buf1, e_buf0, e_buf1, sems):
  s_bufs = (s_buf0, s_buf1)
  e_bufs = (e_buf0, e_buf1)
  wid = lax.axis_index("s") * NC + lax.axis_index("c")
  f = wid // NCHUNK
  ch = wid % NCHUNK
  base = ch * EPT

  pltpu.sync_copy(xaT.at[pl.ds(f * N, N)], xa_col)

  zero = jnp.zeros((L,), jnp.float32)

  @pl.loop(0, N // L)
  def _(i):
    agg_col[pl.ds(i * L, L)] = zero

  def chunk_copies(j, b):
    off = base + j * CB
    return (
        pltpu.make_async_copy(sd.at[pl.ds(off, CB)], s_bufs[b], sems.at[b]),
        pltpu.make_async_copy(ecT.at[pl.ds(f * E + off, CB)], e_bufs[b],
                              sems.at[b]),
    )

  def start(j, b):
    for c in chunk_copies(j, b):
      c.start()

  def wait(j, b):
    for c in chunk_copies(j, b):
      c.wait()

  def process(b):
    sb = s_bufs[b]
    eb = e_bufs[b]

    def one_pass(_):
      def group(i, viol):
        o = i * L
        sdv = sb[pl.ds(o, L)]
        si = sdv & 0xFFFF
        di = lax.shift_right_logical(sdv, 16)
        v = plsc.load_gather(xa_col, [si]) + eb[pl.ds(o, L)]
        cur = plsc.load_gather(agg_col, [di])
        m = v > cur
        plsc.store_scatter(agg_col, [di], v, mask=m)
        cur2 = plsc.load_gather(agg_col, [di])
        return viol | (m & (v > cur2))

      viol = lax.fori_loop(0, CB // L, group,
                           jnp.zeros((L,), jnp.bool_), unroll=4)
      return jnp.any(viol)

    lax.while_loop(lambda go: go, one_pass, jnp.bool_(True))

  start(0, 0)

  @pl.loop(0, NSTEP, step=2)
  def _(jj):
    for b in (0, 1):
      j = jj + b

      @pl.when(j + 1 < NSTEP)
      def _():
        start(j + 1, 1 - b)

      wait(j, b)
      process(b)

  pltpu.sync_copy(agg_col, part.at[pl.ds(wid * N, N)])


_sc_aggregate = functools.partial(
    pl.kernel,
    out_type=jax.ShapeDtypeStruct((NW * N,), jnp.float32),
    mesh=plsc.VectorSubcoreMesh(core_axis_name="c", subcore_axis_name="s"),
    compiler_params=pltpu.CompilerParams(needs_layout_passes=False),
    scratch_types=[
        pltpu.VMEM((N,), jnp.float32),
        pltpu.VMEM((N,), jnp.float32),
        pltpu.VMEM((CB,), jnp.int32),
        pltpu.VMEM((CB,), jnp.int32),
        pltpu.VMEM((CB,), jnp.float32),
        pltpu.VMEM((CB,), jnp.float32),
        pltpu.SemaphoreType.DMA((2,)),
    ],
)(_sc_aggregate_body)


_BR = 1000
_BE = _BR * 16
_GE = E // _BE
_BN = 1024


def _edge_prep_body(ea_ref, w_ref, b_ref, ec_ref):
  ea = ea_ref[...]
  for f in range(FM):
    ec_ref[f] = _mm(ea, w_ref[f]) + b_ref[f]


_edge_prep = pl.pallas_call(
    _edge_prep_body,
    grid=(_GE,),
    in_specs=[
        pl.BlockSpec((_BR, 128), lambda i: (i, 0)),
        pl.BlockSpec((FM, 128, 16), lambda i: (0, 0, 0)),
        pl.BlockSpec((FM, 1), lambda i: (0, 0)),
    ],
    out_specs=pl.BlockSpec((FM, _BR, 16), lambda i: (0, i, 0)),
    out_shape=jax.ShapeDtypeStruct((FM, E // 16, 16), jnp.float32),
)


def _pack_body(sr_ref, dr_ref, sd_ref):
  sd_ref[...] = sr_ref[...] | lax.shift_left(dr_ref[...], 16)


_pack_call = pl.pallas_call(
    _pack_body,
    grid=(1,),
    in_specs=[
        pl.BlockSpec((50, E // 50), lambda i: (0, 0)),
        pl.BlockSpec((50, E // 50), lambda i: (0, 0)),
    ],
    out_specs=pl.BlockSpec((50, E // 50), lambda i: (0, 0)),
    out_shape=jax.ShapeDtypeStruct((50, E // 50), jnp.int32),
)


def _node_prep_body(x_ref, i16_ref, w_ref, xT_ref, xaT_ref):
  xTb = lax.dot_general(i16_ref[...], x_ref[...], (((1,), (1,)), ((), ())),
                        preferred_element_type=jnp.float32,
                        precision=lax.Precision.HIGHEST)
  xT_ref[...] = xTb
  xaT_ref[...] = _mm(w_ref[...], xTb)


_node_prep = pl.pallas_call(
    _node_prep_body,
    grid=(pl.cdiv(N, _BN),),
    in_specs=[
        pl.BlockSpec((_BN, FX), lambda i: (i, 0)),
        pl.BlockSpec((FX, FX), lambda i: (0, 0)),
        pl.BlockSpec((FM, FX), lambda i: (0, 0)),
    ],
    out_specs=[
        pl.BlockSpec((FX, _BN), lambda i: (0, i)),
        pl.BlockSpec((FM, _BN), lambda i: (0, i)),
    ],
    out_shape=[
        jax.ShapeDtypeStruct((FX, N), jnp.float32),
        jax.ShapeDtypeStruct((FM, N), jnp.float32),
    ],
)


def _combine(part_ref, xT_ref, w2ax_ref, w2aa_ref, b2a_ref, w2b_ref, b2b_ref):
  p = part_ref[...]
  aggr = p[:, 0, :]
  for k in range(1, NCHUNK):
    aggr = jnp.maximum(aggr, p[:, k, :])
  xT = xT_ref[...]
  h = _mm(w2ax_ref[...], xT) + _mm(w2aa_ref[...], aggr) + b2a_ref[...]
  h = jnp.maximum(h, 0.0)
  comb = _mm(w2b_ref[...], h) + b2b_ref[...]
  nor = jnp.sqrt(jnp.sum(comb * comb, axis=0, keepdims=True))
  comb = comb / jnp.maximum(1.0, nor)
  return comb, xT[0:8, :]


def _update_body(part_ref, xT_ref, w2ax_ref, w2aa_ref, b2a_ref, w2b_ref,
                 b2b_ref, w1c_ref, w1x_ref, xn_ref, xan_ref):
  comb, x8 = _combine(part_ref, xT_ref, w2ax_ref, w2aa_ref, b2a_ref,
                      w2b_ref, b2b_ref)
  xn_ref[0:8, :] = comb
  xn_ref[8:16, :] = x8
  xan_ref[...] = _mm(w1c_ref[...], comb) + _mm(w1x_ref[...], x8)


_mid_specs = [
    pl.BlockSpec((FM, NCHUNK, _BN), lambda i: (0, 0, i)),
    pl.BlockSpec((FX, _BN), lambda i: (0, i)),
    pl.BlockSpec((FM, FX), lambda i: (0, 0)),
    pl.BlockSpec((FM, FM), lambda i: (0, 0)),
    pl.BlockSpec((FM, 1), lambda i: (0, 0)),
    pl.BlockSpec((8, FM), lambda i: (0, 0)),
    pl.BlockSpec((8, 1), lambda i: (0, 0)),
]

_update_call = pl.pallas_call(
    _update_body,
    grid=(pl.cdiv(N, _BN),),
    in_specs=_mid_specs + [
        pl.BlockSpec((FM, 8), lambda i: (0, 0)),
        pl.BlockSpec((FM, 8), lambda i: (0, 0)),
    ],
    out_specs=[
        pl.BlockSpec((FX, _BN), lambda i: (0, i)),
        pl.BlockSpec((FM, _BN), lambda i: (0, i)),
    ],
    out_shape=[
        jax.ShapeDtypeStruct((FX, N), jnp.float32),
        jax.ShapeDtypeStruct((FM, N), jnp.float32),
    ],
)


def _final_body(part_ref, xT_ref, w2ax_ref, w2aa_ref, b2a_ref, w2b_ref,
                b2b_ref, ea_ref, eb_ref, out_ref):
  comb, x8 = _combine(part_ref, xT_ref, w2ax_ref, w2aa_ref, b2a_ref,
                      w2b_ref, b2b_ref)
  out_ref[...] = _mmt(comb, ea_ref[...]) + _mmt(x8, eb_ref[...])


_final_call = pl.pallas_call(
    _final_body,
    grid=(pl.cdiv(N, _BN),),
    in_specs=_mid_specs + [
        pl.BlockSpec((8, FX), lambda i: (0, 0)),
        pl.BlockSpec((8, FX), lambda i: (0, 0)),
    ],
    out_specs=pl.BlockSpec((_BN, FX), lambda i: (i, 0)),
    out_shape=jax.ShapeDtypeStruct((N, FX), jnp.float32),
)


def kernel(x, edge_index, edge_attr, W1, b1, W2a, b2a, W2b, b2b):
  src = edge_index[0].reshape(50, E // 50)
  dst = edge_index[1].reshape(50, E // 50)

  w4 = jnp.einsum("kf,aA->fakA", W1[FX:], jnp.eye(16, dtype=jnp.float32))
  w4 = w4.reshape(FM, 128, 16)
  ec4 = _edge_prep(edge_attr.reshape(E // 16, 128), w4, b1.reshape(FM, 1))
  ecT = ec4.reshape(FM * E)
  sd = _pack_call(src, dst).reshape(E)

  i16 = jnp.eye(FX, dtype=jnp.float32)
  w1xT = jnp.transpose(W1[:FX])
  xT, xaT = _node_prep(x, i16, w1xT)

  w2ax = jnp.transpose(W2a[:FX])
  w2aa = jnp.transpose(W2a[FX:])
  w2b = jnp.transpose(W2b)
  b2a_c = b2a.reshape(FM, 1)
  b2b_c = b2b.reshape(8, 1)
  w1c = w1xT[:, 0:8]
  w1x = w1xT[:, 8:16]
  e_hi = jnp.eye(8, FX, dtype=jnp.float32)
  e_lo = jnp.eye(8, FX, k=8, dtype=jnp.float32)

  for _ in range(2):
    part = _sc_aggregate(xaT.reshape(FM * N), sd, ecT)
    xT, xaT = _update_call(part.reshape(FM, NCHUNK, N), xT, w2ax, w2aa,
                           b2a_c, w2b, b2b_c, w1c, w1x)

  part = _sc_aggregate(xaT.reshape(FM * N), sd, ecT)
  return _final_call(part.reshape(FM, NCHUNK, N), xT, w2ax, w2aa,
                     b2a_c, w2b, b2b_c, e_hi, e_lo)


# speedup vs baseline: 12.8277x; 1.0346x over previous
import functools

import jax
import jax.numpy as jnp
from jax import lax
from jax.experimental import pallas as pl
from jax.experimental.pallas import tpu as pltpu
from jax.experimental.pallas import tpu_sc as plsc

N = 50000
E = 1600000
FX = 16
FM = 4
NC, NS, L = 2, 16, 16
NW = NC * NS
NCHUNK = NW // FM
EPT = E // NCHUNK
CB = 4000
NSTEP = EPT // CB


def _mm(a, b):
  return jnp.matmul(a, b)


def _mmt(a, b):
  return lax.dot_general(a, b, (((0,), (0,)), ((), ())),
                         preferred_element_type=jnp.float32,
                         precision=lax.Precision.HIGHEST)


def _sc_aggregate_body(xaT, sd, ec0, ec1, ec2, ec3, part, xa_col, agg_col,
                       s_buf0, s_buf1, e_buf0, e_buf1, sems):
  s_bufs = (s_buf0, s_buf1)
  e_bufs = (e_buf0, e_buf1)
  ecs = (ec0, ec1, ec2, ec3)
  wid = lax.axis_index("s") * NC + lax.axis_index("c")
  f = wid // NCHUNK
  ch = wid % NCHUNK
  base = ch * EPT

  pltpu.sync_copy(xaT.at[pl.ds(f * N, N)], xa_col)

  zero = jnp.zeros((L,), jnp.float32)

  @pl.loop(0, N // L, unroll=8)
  def _(i):
    agg_col[pl.ds(i * L, L)] = zero

  def start(j, b):
    off = base + j * CB
    pltpu.make_async_copy(sd.at[pl.ds(off, CB)], s_bufs[b],
                          sems.at[b]).start()
    for ff in range(FM):
      @pl.when(f == ff)
      def _():
        pltpu.make_async_copy(ecs[ff].at[pl.ds(off, CB)], e_bufs[b],
                              sems.at[b]).start()

  def wait(j, b):
    off = base + j * CB
    pltpu.make_async_copy(sd.at[pl.ds(off, CB)], s_bufs[b], sems.at[b]).wait()
    pltpu.make_async_copy(ec0.at[pl.ds(off, CB)], e_bufs[b],
                          sems.at[b]).wait()

  def process(b):
    sb = s_bufs[b]
    eb = e_bufs[b]

    def one_pass(_):
      def group(i, viol):
        o = i * L
        sdv = sb[pl.ds(o, L)]
        si = sdv & 0xFFFF
        di = lax.shift_right_logical(sdv, 16)
        v = plsc.load_gather(xa_col, [si]) + eb[pl.ds(o, L)]
        cur = plsc.load_gather(agg_col, [di])
        plsc.store_scatter(agg_col, [di], v, mask=v > cur)
        cur2 = plsc.load_gather(agg_col, [di])
        return viol | (v > cur2)

      viol = lax.fori_loop(0, CB // L, group,
                           jnp.zeros((L,), jnp.bool_), unroll=8)
      return jnp.any(viol)

    lax.while_loop(lambda go: go, one_pass, jnp.bool_(True))

  start(0, 0)

  @pl.loop(0, NSTEP, step=2)
  def _(jj):
    for b in (0, 1):
      j = jj + b

      @pl.when(j + 1 < NSTEP)
      def _():
        start(j + 1, 1 - b)

      wait(j, b)
      process(b)

  pltpu.sync_copy(agg_col, part.at[pl.ds(wid * N, N)])


_sc_aggregate = functools.partial(
    pl.kernel,
    out_type=jax.ShapeDtypeStruct((NW * N,), jnp.float32),
    mesh=plsc.VectorSubcoreMesh(core_axis_name="c", subcore_axis_name="s"),
    compiler_params=pltpu.CompilerParams(needs_layout_passes=False),
    scratch_types=[
        pltpu.VMEM((N,), jnp.float32),
        pltpu.VMEM((N,), jnp.float32),
        pltpu.VMEM((CB,), jnp.int32),
        pltpu.VMEM((CB,), jnp.int32),
        pltpu.VMEM((CB,), jnp.float32),
        pltpu.VMEM((CB,), jnp.float32),
        pltpu.SemaphoreType.DMA((2,)),
    ],
)(_sc_aggregate_body)


_GP = 20
_PR = E // 16 // _GP
_PN = 2560
_BN = 1024


def _prep_body(ea_ref, sr_ref, dr_ref, x_ref, w4_ref, b_ref, i16_ref, w1_ref,
               ec0_ref, ec1_ref, ec2_ref, ec3_ref, sd_ref, xT_ref, xaT_ref):
  ea = ea_ref[...]
  ec_refs = (ec0_ref, ec1_ref, ec2_ref, ec3_ref)
  for f in range(FM):
    ec_refs[f][...] = _mm(ea, w4_ref[f]) + b_ref[f]
  sd_ref[...] = sr_ref[...] | lax.shift_left(dr_ref[...], 16)
  xTb = lax.dot_general(i16_ref[...], x_ref[...], (((1,), (1,)), ((), ())),
                        preferred_element_type=jnp.float32,
                        precision=lax.Precision.HIGHEST)
  xT_ref[...] = xTb
  xaT_ref[...] = _mm(w1_ref[...], xTb)


_prep_call = pl.pallas_call(
    _prep_body,
    grid=(_GP,),
    in_specs=[
        pl.BlockSpec((_PR, 128), lambda i: (i, 0)),
        pl.BlockSpec((_PR, 16), lambda i: (i, 0)),
        pl.BlockSpec((_PR, 16), lambda i: (i, 0)),
        pl.BlockSpec((_PN, FX), lambda i: (i, 0)),
        pl.BlockSpec((FM, 128, 16), lambda i: (0, 0, 0)),
        pl.BlockSpec((FM, 1), lambda i: (0, 0)),
        pl.BlockSpec((FX, FX), lambda i: (0, 0)),
        pl.BlockSpec((FM, FX), lambda i: (0, 0)),
    ],
    out_specs=[
        pl.BlockSpec((_PR, 16), lambda i: (i, 0)),
        pl.BlockSpec((_PR, 16), lambda i: (i, 0)),
        pl.BlockSpec((_PR, 16), lambda i: (i, 0)),
        pl.BlockSpec((_PR, 16), lambda i: (i, 0)),
        pl.BlockSpec((_PR, 16), lambda i: (i, 0)),
        pl.BlockSpec((FX, _PN), lambda i: (0, i)),
        pl.BlockSpec((FM, _PN), lambda i: (0, i)),
    ],
    out_shape=[
        jax.ShapeDtypeStruct((E // 16, 16), jnp.float32),
        jax.ShapeDtypeStruct((E // 16, 16), jnp.float32),
        jax.ShapeDtypeStruct((E // 16, 16), jnp.float32),
        jax.ShapeDtypeStruct((E // 16, 16), jnp.float32),
        jax.ShapeDtypeStruct((E // 16, 16), jnp.int32),
        jax.ShapeDtypeStruct((FX, N), jnp.float32),
        jax.ShapeDtypeStruct((FM, N), jnp.float32),
    ],
)


def _combine(part_ref, xT_ref, w2ax_ref, w2aa_ref, b2a_ref, w2b_ref, b2b_ref):
  p = part_ref[...]
  aggr = p[:, 0, :]
  for k in range(1, NCHUNK):
    aggr = jnp.maximum(aggr, p[:, k, :])
  xT = xT_ref[...]
  h = _mm(w2ax_ref[...], xT) + _mm(w2aa_ref[...], aggr) + b2a_ref[...]
  h = jnp.maximum(h, 0.0)
  comb = _mm(w2b_ref[...], h) + b2b_ref[...]
  nor = jnp.sqrt(jnp.sum(comb * comb, axis=0, keepdims=True))
  comb = comb / jnp.maximum(1.0, nor)
  return comb, xT[0:8, :]


def _update_body(part_ref, xT_ref, w2ax_ref, w2aa_ref, b2a_ref, w2b_ref,
                 b2b_ref, w1c_ref, w1x_ref, xn_ref, xan_ref):
  comb, x8 = _combine(part_ref, xT_ref, w2ax_ref, w2aa_ref, b2a_ref,
                      w2b_ref, b2b_ref)
  xn_ref[0:8, :] = comb
  xn_ref[8:16, :] = x8
  xan_ref[...] = _mm(w1c_ref[...], comb) + _mm(w1x_ref[...], x8)


_mid_specs = [
    pl.BlockSpec((FM, NCHUNK, _BN), lambda i: (0, 0, i)),
    pl.BlockSpec((FX, _BN), lambda i: (0, i)),
    pl.BlockSpec((FM, FX), lambda i: (0, 0)),
    pl.BlockSpec((FM, FM), lambda i: (0, 0)),
    pl.BlockSpec((FM, 1), lambda i: (0, 0)),
    pl.BlockSpec((8, FM), lambda i: (0, 0)),
    pl.BlockSpec((8, 1), lambda i: (0, 0)),
]

_update_call = pl.pallas_call(
    _update_body,
    grid=(pl.cdiv(N, _BN),),
    in_specs=_mid_specs + [
        pl.BlockSpec((FM, 8), lambda i: (0, 0)),
        pl.BlockSpec((FM, 8), lambda i: (0, 0)),
    ],
    out_specs=[
        pl.BlockSpec((FX, _BN), lambda i: (0, i)),
        pl.BlockSpec((FM, _BN), lambda i: (0, i)),
    ],
    out_shape=[
        jax.ShapeDtypeStruct((FX, N), jnp.float32),
        jax.ShapeDtypeStruct((FM, N), jnp.float32),
    ],
)


def _final_body(part_ref, xT_ref, w2ax_ref, w2aa_ref, b2a_ref, w2b_ref,
                b2b_ref, ea_ref, eb_ref, out_ref):
  comb, x8 = _combine(part_ref, xT_ref, w2ax_ref, w2aa_ref, b2a_ref,
                      w2b_ref, b2b_ref)
  out_ref[...] = _mmt(comb, ea_ref[...]) + _mmt(x8, eb_ref[...])


_final_call = pl.pallas_call(
    _final_body,
    grid=(pl.cdiv(N, _BN),),
    in_specs=_mid_specs + [
        pl.BlockSpec((8, FX), lambda i: (0, 0)),
        pl.BlockSpec((8, FX), lambda i: (0, 0)),
    ],
    out_specs=pl.BlockSpec((_BN, FX), lambda i: (i, 0)),
    out_shape=jax.ShapeDtypeStruct((N, FX), jnp.float32),
)


def kernel(x, edge_index, edge_attr, W1, b1, W2a, b2a, W2b, b2b):
  src = edge_index[0].reshape(E // 16, 16)
  dst = edge_index[1].reshape(E // 16, 16)

  w4 = jnp.einsum("kf,aA->fakA", W1[FX:], jnp.eye(16, dtype=jnp.float32))
  w4 = w4.reshape(FM, 128, 16)
  i16 = jnp.eye(FX, dtype=jnp.float32)
  w1xT = jnp.transpose(W1[:FX])

  ec0, ec1, ec2, ec3, sd, xT, xaT = _prep_call(
      edge_attr.reshape(E // 16, 128), src, dst, x, w4, b1.reshape(FM, 1),
      i16, w1xT)
  ecs = (ec0.reshape(E), ec1.reshape(E), ec2.reshape(E), ec3.reshape(E))
  sd = sd.reshape(E)

  w2ax = jnp.transpose(W2a[:FX])
  w2aa = jnp.transpose(W2a[FX:])
  w2b = jnp.transpose(W2b)
  b2a_c = b2a.reshape(FM, 1)
  b2b_c = b2b.reshape(8, 1)
  w1c = w1xT[:, 0:8]
  w1x = w1xT[:, 8:16]
  e_hi = jnp.eye(8, FX, dtype=jnp.float32)
  e_lo = jnp.eye(8, FX, k=8, dtype=jnp.float32)

  for _ in range(2):
    part = _sc_aggregate(xaT.reshape(FM * N), sd, *ecs)
    xT, xaT = _update_call(part.reshape(FM, NCHUNK, N), xT, w2ax, w2aa,
                           b2a_c, w2b, b2b_c, w1c, w1x)

  part = _sc_aggregate(xaT.reshape(FM * N), sd, *ecs)
  return _final_call(part.reshape(FM, NCHUNK, N), xT, w2ax, w2aa,
                     b2a_c, w2b, b2b_c, e_hi, e_lo)


# speedup vs baseline: 22.7755x; 1.7755x over previous
import functools

import jax
import jax.numpy as jnp
from jax import lax
from jax.experimental import pallas as pl
from jax.experimental.pallas import tpu as pltpu
from jax.experimental.pallas import tpu_sc as plsc

N = 50000
E = 1600000
FX = 16
FM = 4
NC, NS, L = 2, 16, 16
NW = NC * NS
NCHUNK = NW // FM
EPT = E // NCHUNK
CB = 4000
NSTEP = EPT // CB


def _mm(a, b):
  return jnp.matmul(a, b)


def _sc_aggregate_body(xa0, xa1, xa2, xa3, sd, ec0, ec1, ec2, ec3, part,
                       xa_col, agg_col, s_buf0, s_buf1, e_buf0, e_buf1, sems):
  s_bufs = (s_buf0, s_buf1)
  e_bufs = (e_buf0, e_buf1)
  xas = (xa0, xa1, xa2, xa3)
  ecs = (ec0, ec1, ec2, ec3)
  wid = lax.axis_index("s") * NC + lax.axis_index("c")
  f = wid // NCHUNK
  ch = wid % NCHUNK
  base = ch * EPT

  for ff in range(FM):
    @pl.when(f == ff)
    def _():
      pltpu.sync_copy(xas[ff], xa_col)

  zero = jnp.zeros((L,), jnp.float32)

  @pl.loop(0, N // L, unroll=8)
  def _(i):
    agg_col[pl.ds(i * L, L)] = zero

  def start(j, b):
    off = base + j * CB
    pltpu.make_async_copy(sd.at[pl.ds(off, CB)], s_bufs[b],
                          sems.at[b]).start()
    for ff in range(FM):
      @pl.when(f == ff)
      def _():
        pltpu.make_async_copy(ecs[ff].at[pl.ds(off, CB)], e_bufs[b],
                              sems.at[b]).start()

  def wait(j, b):
    off = base + j * CB
    pltpu.make_async_copy(sd.at[pl.ds(off, CB)], s_bufs[b], sems.at[b]).wait()
    pltpu.make_async_copy(ec0.at[pl.ds(off, CB)], e_bufs[b],
                          sems.at[b]).wait()

  def process(b):
    sb = s_bufs[b]
    eb = e_bufs[b]

    def one_pass(_):
      def group(i, viol):
        o = i * L
        sdv = sb[pl.ds(o, L)]
        si = sdv & 0xFFFF
        di = lax.shift_right_logical(sdv, 16)
        v = plsc.load_gather(xa_col, [si]) + eb[pl.ds(o, L)]
        cur = plsc.load_gather(agg_col, [di])
        plsc.store_scatter(agg_col, [di], v, mask=v > cur)
        cur2 = plsc.load_gather(agg_col, [di])
        return viol | (v > cur2)

      viol = lax.fori_loop(0, CB // L, group,
                           jnp.zeros((L,), jnp.bool_), unroll=8)
      return jnp.any(viol)

    lax.while_loop(lambda go: go, one_pass, jnp.bool_(True))

  start(0, 0)

  @pl.loop(0, NSTEP, step=2)
  def _(jj):
    for b in (0, 1):
      j = jj + b

      @pl.when(j + 1 < NSTEP)
      def _():
        start(j + 1, 1 - b)

      wait(j, b)
      process(b)

  pltpu.sync_copy(agg_col, part.at[pl.ds(wid * N, N)])


_sc_aggregate = functools.partial(
    pl.kernel,
    out_type=jax.ShapeDtypeStruct((NW * N,), jnp.float32),
    mesh=plsc.VectorSubcoreMesh(core_axis_name="c", subcore_axis_name="s"),
    compiler_params=pltpu.CompilerParams(needs_layout_passes=False),
    scratch_types=[
        pltpu.VMEM((N,), jnp.float32),
        pltpu.VMEM((N,), jnp.float32),
        pltpu.VMEM((CB,), jnp.int32),
        pltpu.VMEM((CB,), jnp.int32),
        pltpu.VMEM((CB,), jnp.float32),
        pltpu.VMEM((CB,), jnp.float32),
        pltpu.SemaphoreType.DMA((2,)),
    ],
)(_sc_aggregate_body)


_PE = 81920
_GP = -(-E // _PE)
_PN = 3072
_GN = -(-N // _PN)
_BN = 1024


def _prep_body(eaT_ref, sr_ref, dr_ref, w1e_ref, b_ref,
               ec0_ref, ec1_ref, ec2_ref, ec3_ref, sd_ref):
  ecm = _mm(w1e_ref[...], eaT_ref[...]) + b_ref[...]
  ec_refs = (ec0_ref, ec1_ref, ec2_ref, ec3_ref)
  for f in range(FM):
    ec_refs[f][...] = ecm[f]
  sd_ref[...] = sr_ref[...] | lax.shift_left(dr_ref[...], 16)


_prep_call = pl.pallas_call(
    _prep_body,
    grid=(_GP,),
    in_specs=[
        pl.BlockSpec((8, _PE), lambda i: (0, i)),
        pl.BlockSpec((_PE,), lambda i: (i,)),
        pl.BlockSpec((_PE,), lambda i: (i,)),
        pl.BlockSpec((FM, 8), lambda i: (0, 0)),
        pl.BlockSpec((FM, 1), lambda i: (0, 0)),
    ],
    out_specs=[
        pl.BlockSpec((_PE,), lambda i: (i,)),
        pl.BlockSpec((_PE,), lambda i: (i,)),
        pl.BlockSpec((_PE,), lambda i: (i,)),
        pl.BlockSpec((_PE,), lambda i: (i,)),
        pl.BlockSpec((_PE,), lambda i: (i,)),
    ],
    out_shape=[
        jax.ShapeDtypeStruct((E,), jnp.float32),
        jax.ShapeDtypeStruct((E,), jnp.float32),
        jax.ShapeDtypeStruct((E,), jnp.float32),
        jax.ShapeDtypeStruct((E,), jnp.float32),
        jax.ShapeDtypeStruct((E,), jnp.int32),
    ],
)


def _nprep_body(xT_ref, w1_ref, xa0_ref, xa1_ref, xa2_ref, xa3_ref):
  xab = _mm(w1_ref[...], xT_ref[...])
  xa_refs = (xa0_ref, xa1_ref, xa2_ref, xa3_ref)
  for f in range(FM):
    xa_refs[f][...] = xab[f]


_nprep_call = pl.pallas_call(
    _nprep_body,
    grid=(_GN,),
    in_specs=[
        pl.BlockSpec((FX, _PN), lambda i: (0, i)),
        pl.BlockSpec((FM, FX), lambda i: (0, 0)),
    ],
    out_specs=[
        pl.BlockSpec((_PN,), lambda i: (i,)),
        pl.BlockSpec((_PN,), lambda i: (i,)),
        pl.BlockSpec((_PN,), lambda i: (i,)),
        pl.BlockSpec((_PN,), lambda i: (i,)),
    ],
    out_shape=[
        jax.ShapeDtypeStruct((N,), jnp.float32),
        jax.ShapeDtypeStruct((N,), jnp.float32),
        jax.ShapeDtypeStruct((N,), jnp.float32),
        jax.ShapeDtypeStruct((N,), jnp.float32),
    ],
)


def _combine(part_ref, xT_ref, w2ax_ref, w2aa_ref, b2a_ref, w2b_ref, b2b_ref):
  p = part_ref[...]
  aggr = p[:, 0, :]
  for k in range(1, NCHUNK):
    aggr = jnp.maximum(aggr, p[:, k, :])
  xT = xT_ref[...]
  h = _mm(w2ax_ref[...], xT) + _mm(w2aa_ref[...], aggr) + b2a_ref[...]
  h = jnp.maximum(h, 0.0)
  comb = _mm(w2b_ref[...], h) + b2b_ref[...]
  nor = jnp.sqrt(jnp.sum(comb * comb, axis=0, keepdims=True))
  comb = comb / jnp.maximum(1.0, nor)
  return comb, xT[0:8, :]


def _update_body(part_ref, xT_ref, w2ax_ref, w2aa_ref, b2a_ref, w2b_ref,
                 b2b_ref, w1c_ref, w1x_ref, xn_ref,
                 xa0_ref, xa1_ref, xa2_ref, xa3_ref):
  comb, x8 = _combine(part_ref, xT_ref, w2ax_ref, w2aa_ref, b2a_ref,
                      w2b_ref, b2b_ref)
  xn_ref[0:8, :] = comb
  xn_ref[8:16, :] = x8
  xan = _mm(w1c_ref[...], comb) + _mm(w1x_ref[...], x8)
  xa_refs = (xa0_ref, xa1_ref, xa2_ref, xa3_ref)
  for f in range(FM):
    xa_refs[f][...] = xan[f]


_mid_specs = [
    pl.BlockSpec((FM, NCHUNK, _BN), lambda i: (0, 0, i)),
    pl.BlockSpec((FX, _BN), lambda i: (0, i)),
    pl.BlockSpec((FM, FX), lambda i: (0, 0)),
    pl.BlockSpec((FM, FM), lambda i: (0, 0)),
    pl.BlockSpec((FM, 1), lambda i: (0, 0)),
    pl.BlockSpec((8, FM), lambda i: (0, 0)),
    pl.BlockSpec((8, 1), lambda i: (0, 0)),
]

_update_call = pl.pallas_call(
    _update_body,
    grid=(pl.cdiv(N, _BN),),
    in_specs=_mid_specs + [
        pl.BlockSpec((FM, 8), lambda i: (0, 0)),
        pl.BlockSpec((FM, 8), lambda i: (0, 0)),
    ],
    out_specs=[
        pl.BlockSpec((FX, _BN), lambda i: (0, i)),
        pl.BlockSpec((_BN,), lambda i: (i,)),
        pl.BlockSpec((_BN,), lambda i: (i,)),
        pl.BlockSpec((_BN,), lambda i: (i,)),
        pl.BlockSpec((_BN,), lambda i: (i,)),
    ],
    out_shape=[
        jax.ShapeDtypeStruct((FX, N), jnp.float32),
        jax.ShapeDtypeStruct((N,), jnp.float32),
        jax.ShapeDtypeStruct((N,), jnp.float32),
        jax.ShapeDtypeStruct((N,), jnp.float32),
        jax.ShapeDtypeStruct((N,), jnp.float32),
    ],
)


def _final_body(part_ref, xT_ref, w2ax_ref, w2aa_ref, b2a_ref, w2b_ref,
                b2b_ref, xn_ref):
  comb, x8 = _combine(part_ref, xT_ref, w2ax_ref, w2aa_ref, b2a_ref,
                      w2b_ref, b2b_ref)
  xn_ref[0:8, :] = comb
  xn_ref[8:16, :] = x8


_final_call = pl.pallas_call(
    _final_body,
    grid=(pl.cdiv(N, _BN),),
    in_specs=_mid_specs,
    out_specs=pl.BlockSpec((FX, _BN), lambda i: (0, i)),
    out_shape=jax.ShapeDtypeStruct((FX, N), jnp.float32),
)


def kernel(x, edge_index, edge_attr, W1, b1, W2a, b2a, W2b, b2b):
  eaT = jnp.transpose(edge_attr)
  xT = jnp.transpose(x)
  src = edge_index[0]
  dst = edge_index[1]

  w1eT = jnp.transpose(W1[FX:])
  w1xT = jnp.transpose(W1[:FX])

  ec0, ec1, ec2, ec3, sd = _prep_call(eaT, src, dst, w1eT,
                                      b1.reshape(FM, 1))
  xas = _nprep_call(xT, w1xT)

  w2ax = jnp.transpose(W2a[:FX])
  w2aa = jnp.transpose(W2a[FX:])
  w2b = jnp.transpose(W2b)
  b2a_c = b2a.reshape(FM, 1)
  b2b_c = b2b.reshape(8, 1)
  w1c = w1xT[:, 0:8]
  w1x = w1xT[:, 8:16]

  for _ in range(2):
    part = _sc_aggregate(*xas, sd, ec0, ec1, ec2, ec3)
    xT, *xas = _update_call(part.reshape(FM, NCHUNK, N), xT, w2ax, w2aa,
                            b2a_c, w2b, b2b_c, w1c, w1x)

  part = _sc_aggregate(*xas, sd, ec0, ec1, ec2, ec3)
  xnT = _final_call(part.reshape(FM, NCHUNK, N), xT, w2ax, w2aa,
                    b2a_c, w2b, b2b_c)
  return jnp.transpose(xnT)


# speedup vs baseline: 26.6267x; 1.1691x over previous
import functools

import jax
import jax.numpy as jnp
from jax import lax
from jax.experimental import pallas as pl
from jax.experimental.pallas import tpu as pltpu
from jax.experimental.pallas import tpu_sc as plsc

N = 50000
E = 1600000
FX = 16
FM = 4
NC, NS, L = 2, 16, 16
NW = NC * NS
NCHUNK = NW // FM
EPT = E // NCHUNK
CB = 2000
NSTEP = EPT // CB


def _mm(a, b):
  return jnp.matmul(a, b)


def _sc_aggregate_body(xa0, xa1, xa2, xa3, sd, ec0, ec1, ec2, ec3, part,
                       xa_col, agg_col, s_buf0, s_buf1, e_buf0, e_buf1, sems):
  s_bufs = (s_buf0, s_buf1)
  e_bufs = (e_buf0, e_buf1)
  xas = (xa0, xa1, xa2, xa3)
  ecs = (ec0, ec1, ec2, ec3)
  wid = lax.axis_index("s") * NC + lax.axis_index("c")
  f = wid // NCHUNK
  ch = wid % NCHUNK
  base = ch * EPT

  for ff in range(FM):
    @pl.when(f == ff)
    def _():
      pltpu.sync_copy(xas[ff], xa_col)

  zero = jnp.zeros((L,), jnp.float32)

  @pl.loop(0, N // L, unroll=8)
  def _(i):
    agg_col[pl.ds(i * L, L)] = zero

  def start(j, b):
    off = base + j * CB
    pltpu.make_async_copy(sd.at[pl.ds(off, CB)], s_bufs[b],
                          sems.at[b]).start()
    for ff in range(FM):
      @pl.when(f == ff)
      def _():
        pltpu.make_async_copy(ecs[ff].at[pl.ds(off, CB)], e_bufs[b],
                              sems.at[b]).start()

  def wait(j, b):
    off = base + j * CB
    pltpu.make_async_copy(sd.at[pl.ds(off, CB)], s_bufs[b], sems.at[b]).wait()
    pltpu.make_async_copy(ec0.at[pl.ds(off, CB)], e_bufs[b],
                          sems.at[b]).wait()

  def process(b):
    sb = s_bufs[b]
    eb = e_bufs[b]

    def edge_vec(i):
      o = i * L
      sdv = sb[pl.ds(o, L)]
      si = sdv & 0xFFFF
      di = lax.shift_right_logical(sdv, 16)
      v = plsc.load_gather(xa_col, [si]) + eb[pl.ds(o, L)]
      return di, v

    def scatter_pass():
      def group(i, c):
        di, v = edge_vec(i)
        cur = plsc.load_gather(agg_col, [di])
        plsc.store_scatter(agg_col, [di], v, mask=v > cur)
        return c

      lax.fori_loop(0, CB // L, group, jnp.int32(0), unroll=8)

    def verify_pass():
      def group(i, viol):
        di, v = edge_vec(i)
        cur = plsc.load_gather(agg_col, [di])
        return viol | (v > cur)

      viol = lax.fori_loop(0, CB // L, group,
                           jnp.zeros((L,), jnp.bool_), unroll=8)
      return jnp.any(viol)

    def redo(_):
      scatter_pass()
      return verify_pass()

    scatter_pass()
    lax.while_loop(lambda go: go, redo, verify_pass())

  start(0, 0)

  @pl.loop(0, NSTEP, step=2)
  def _(jj):
    for b in (0, 1):
      j = jj + b

      @pl.when(j + 1 < NSTEP)
      def _():
        start(j + 1, 1 - b)

      wait(j, b)
      process(b)

  pltpu.sync_copy(agg_col, part.at[pl.ds(wid * N, N)])


_sc_aggregate = functools.partial(
    pl.kernel,
    out_type=jax.ShapeDtypeStruct((NW * N,), jnp.float32),
    mesh=plsc.VectorSubcoreMesh(core_axis_name="c", subcore_axis_name="s"),
    compiler_params=pltpu.CompilerParams(needs_layout_passes=False),
    scratch_types=[
        pltpu.VMEM((N,), jnp.float32),
        pltpu.VMEM((N,), jnp.float32),
        pltpu.VMEM((CB,), jnp.int32),
        pltpu.VMEM((CB,), jnp.int32),
        pltpu.VMEM((CB,), jnp.float32),
        pltpu.VMEM((CB,), jnp.float32),
        pltpu.SemaphoreType.DMA((2,)),
    ],
)(_sc_aggregate_body)


_PE = 81920
_GP = -(-E // _PE)
_PN = 3072
_GN = -(-N // _PN)
_BN = 1024


def _prep_body(eaT_ref, sr_ref, dr_ref, w1e_ref, b_ref,
               ec0_ref, ec1_ref, ec2_ref, ec3_ref, sd_ref):
  ecm = _mm(w1e_ref[...], eaT_ref[...]) + b_ref[...]
  ec_refs = (ec0_ref, ec1_ref, ec2_ref, ec3_ref)
  for f in range(FM):
    ec_refs[f][...] = ecm[f]
  sd_ref[...] = sr_ref[...] | lax.shift_left(dr_ref[...], 16)


_prep_call = pl.pallas_call(
    _prep_body,
    grid=(_GP,),
    in_specs=[
        pl.BlockSpec((8, _PE), lambda i: (0, i)),
        pl.BlockSpec((_PE,), lambda i: (i,)),
        pl.BlockSpec((_PE,), lambda i: (i,)),
        pl.BlockSpec((FM, 8), lambda i: (0, 0)),
        pl.BlockSpec((FM, 1), lambda i: (0, 0)),
    ],
    out_specs=[
        pl.BlockSpec((_PE,), lambda i: (i,)),
        pl.BlockSpec((_PE,), lambda i: (i,)),
        pl.BlockSpec((_PE,), lambda i: (i,)),
        pl.BlockSpec((_PE,), lambda i: (i,)),
        pl.BlockSpec((_PE,), lambda i: (i,)),
    ],
    out_shape=[
        jax.ShapeDtypeStruct((E,), jnp.float32),
        jax.ShapeDtypeStruct((E,), jnp.float32),
        jax.ShapeDtypeStruct((E,), jnp.float32),
        jax.ShapeDtypeStruct((E,), jnp.float32),
        jax.ShapeDtypeStruct((E,), jnp.int32),
    ],
)


def _nprep_body(xT_ref, w1_ref, xa0_ref, xa1_ref, xa2_ref, xa3_ref):
  xab = _mm(w1_ref[...], xT_ref[...])
  xa_refs = (xa0_ref, xa1_ref, xa2_ref, xa3_ref)
  for f in range(FM):
    xa_refs[f][...] = xab[f]


_nprep_call = pl.pallas_call(
    _nprep_body,
    grid=(_GN,),
    in_specs=[
        pl.BlockSpec((FX, _PN), lambda i: (0, i)),
        pl.BlockSpec((FM, FX), lambda i: (0, 0)),
    ],
    out_specs=[
        pl.BlockSpec((_PN,), lambda i: (i,)),
        pl.BlockSpec((_PN,), lambda i: (i,)),
        pl.BlockSpec((_PN,), lambda i: (i,)),
        pl.BlockSpec((_PN,), lambda i: (i,)),
    ],
    out_shape=[
        jax.ShapeDtypeStruct((N,), jnp.float32),
        jax.ShapeDtypeStruct((N,), jnp.float32),
        jax.ShapeDtypeStruct((N,), jnp.float32),
        jax.ShapeDtypeStruct((N,), jnp.float32),
    ],
)


def _combine(part_ref, xT_ref, w2ax_ref, w2aa_ref, b2a_ref, w2b_ref, b2b_ref):
  p = part_ref[...]
  aggr = p[:, 0, :]
  for k in range(1, NCHUNK):
    aggr = jnp.maximum(aggr, p[:, k, :])
  xT = xT_ref[...]
  h = _mm(w2ax_ref[...], xT) + _mm(w2aa_ref[...], aggr) + b2a_ref[...]
  h = jnp.maximum(h, 0.0)
  comb = _mm(w2b_ref[...], h) + b2b_ref[...]
  nor = jnp.sqrt(jnp.sum(comb * comb, axis=0, keepdims=True))
  comb = comb / jnp.maximum(1.0, nor)
  return comb, xT[0:8, :]


def _update_body(part_ref, xT_ref, w2ax_ref, w2aa_ref, b2a_ref, w2b_ref,
                 b2b_ref, w1c_ref, w1x_ref, xn_ref,
                 xa0_ref, xa1_ref, xa2_ref, xa3_ref):
  comb, x8 = _combine(part_ref, xT_ref, w2ax_ref, w2aa_ref, b2a_ref,
                      w2b_ref, b2b_ref)
  xn_ref[0:8, :] = comb
  xn_ref[8:16, :] = x8
  xan = _mm(w1c_ref[...], comb) + _mm(w1x_ref[...], x8)
  xa_refs = (xa0_ref, xa1_ref, xa2_ref, xa3_ref)
  for f in range(FM):
    xa_refs[f][...] = xan[f]


_mid_specs = [
    pl.BlockSpec((FM, NCHUNK, _BN), lambda i: (0, 0, i)),
    pl.BlockSpec((FX, _BN), lambda i: (0, i)),
    pl.BlockSpec((FM, FX), lambda i: (0, 0)),
    pl.BlockSpec((FM, FM), lambda i: (0, 0)),
    pl.BlockSpec((FM, 1), lambda i: (0, 0)),
    pl.BlockSpec((8, FM), lambda i: (0, 0)),
    pl.BlockSpec((8, 1), lambda i: (0, 0)),
]

_update_call = pl.pallas_call(
    _update_body,
    grid=(pl.cdiv(N, _BN),),
    in_specs=_mid_specs + [
        pl.BlockSpec((FM, 8), lambda i: (0, 0)),
        pl.BlockSpec((FM, 8), lambda i: (0, 0)),
    ],
    out_specs=[
        pl.BlockSpec((FX, _BN), lambda i: (0, i)),
        pl.BlockSpec((_BN,), lambda i: (i,)),
        pl.BlockSpec((_BN,), lambda i: (i,)),
        pl.BlockSpec((_BN,), lambda i: (i,)),
        pl.BlockSpec((_BN,), lambda i: (i,)),
    ],
    out_shape=[
        jax.ShapeDtypeStruct((FX, N), jnp.float32),
        jax.ShapeDtypeStruct((N,), jnp.float32),
        jax.ShapeDtypeStruct((N,), jnp.float32),
        jax.ShapeDtypeStruct((N,), jnp.float32),
        jax.ShapeDtypeStruct((N,), jnp.float32),
    ],
)


def _final_body(part_ref, xT_ref, w2ax_ref, w2aa_ref, b2a_ref, w2b_ref,
                b2b_ref, xn_ref):
  comb, x8 = _combine(part_ref, xT_ref, w2ax_ref, w2aa_ref, b2a_ref,
                      w2b_ref, b2b_ref)
  xn_ref[0:8, :] = comb
  xn_ref[8:16, :] = x8


_final_call = pl.pallas_call(
    _final_body,
    grid=(pl.cdiv(N, _BN),),
    in_specs=_mid_specs,
    out_specs=pl.BlockSpec((FX, _BN), lambda i: (0, i)),
    out_shape=jax.ShapeDtypeStruct((FX, N), jnp.float32),
)


def kernel(x, edge_index, edge_attr, W1, b1, W2a, b2a, W2b, b2b):
  eaT = jnp.transpose(edge_attr)
  xT = jnp.transpose(x)
  src = edge_index[0]
  dst = edge_index[1]

  w1eT = jnp.transpose(W1[FX:])
  w1xT = jnp.transpose(W1[:FX])

  ec0, ec1, ec2, ec3, sd = _prep_call(eaT, src, dst, w1eT,
                                      b1.reshape(FM, 1))
  xas = _nprep_call(xT, w1xT)

  w2ax = jnp.transpose(W2a[:FX])
  w2aa = jnp.transpose(W2a[FX:])
  w2b = jnp.transpose(W2b)
  b2a_c = b2a.reshape(FM, 1)
  b2b_c = b2b.reshape(8, 1)
  w1c = w1xT[:, 0:8]
  w1x = w1xT[:, 8:16]

  for _ in range(2):
    part = _sc_aggregate(*xas, sd, ec0, ec1, ec2, ec3)
    xT, *xas = _update_call(part.reshape(FM, NCHUNK, N), xT, w2ax, w2aa,
                            b2a_c, w2b, b2b_c, w1c, w1x)

  part = _sc_aggregate(*xas, sd, ec0, ec1, ec2, ec3)
  xnT = _final_call(part.reshape(FM, NCHUNK, N), xT, w2ax, w2aa,
                    b2a_c, w2b, b2b_c)
  return jnp.transpose(xnT)


# speedup vs baseline: 44.1562x; 1.6583x over previous
import functools

import jax
import jax.numpy as jnp
from jax import lax
from jax.experimental import pallas as pl
from jax.experimental.pallas import tpu as pltpu
from jax.experimental.pallas import tpu_sc as plsc

N = 50000
E = 1600000
FX = 16
FM = 4
NC, NS, L = 2, 16, 16
NW = NC * NS
NCHUNK = NW // FM
EPT = E // NCHUNK
CB = 2000
NSTEP = EPT // CB


def _mm(a, b):
  return jnp.matmul(a, b)


def _sc_aggregate_body(xa0, xa1, xa2, xa3, sd, ec0, ec1, ec2, ec3, part,
                       xa_col, agg_col, s_buf0, s_buf1, e_buf0, e_buf1, sems):
  s_bufs = (s_buf0, s_buf1)
  e_bufs = (e_buf0, e_buf1)
  xas = (xa0, xa1, xa2, xa3)
  ecs = (ec0, ec1, ec2, ec3)
  wid = lax.axis_index("s") * NC + lax.axis_index("c")
  f = wid // NCHUNK
  ch = wid % NCHUNK
  base = ch * EPT

  for ff in range(FM):
    @pl.when(f == ff)
    def _():
      pltpu.sync_copy(xas[ff], xa_col)

  zero = jnp.zeros((L,), jnp.float32)

  @pl.loop(0, N // L, unroll=8)
  def _(i):
    agg_col[pl.ds(i * L, L)] = zero

  def start(j, b):
    off = base + j * CB
    pltpu.make_async_copy(sd.at[pl.ds(off, CB)], s_bufs[b],
                          sems.at[b]).start()
    for ff in range(FM):
      @pl.when(f == ff)
      def _():
        pltpu.make_async_copy(ecs[ff].at[pl.ds(off, CB)], e_bufs[b],
                              sems.at[b]).start()

  def wait(j, b):
    off = base + j * CB
    pltpu.make_async_copy(sd.at[pl.ds(off, CB)], s_bufs[b], sems.at[b]).wait()
    pltpu.make_async_copy(ec0.at[pl.ds(off, CB)], e_bufs[b],
                          sems.at[b]).wait()

  def process(b):
    sb = s_bufs[b]
    eb = e_bufs[b]

    def edge_vec(i):
      o = i * L
      sdv = sb[pl.ds(o, L)]
      si = sdv & 0xFFFF
      di = lax.shift_right_logical(sdv, 16)
      v = plsc.load_gather(xa_col, [si]) + eb[pl.ds(o, L)]
      return di, v

    def scatter_pass():
      @plsc.parallel_loop(0, CB // L, unroll=8)
      def _(i):
        di, v = edge_vec(i)
        cur = plsc.load_gather(agg_col, [di])
        plsc.store_scatter(agg_col, [di], v, mask=v > cur)

    def verify_pass():
      @plsc.parallel_loop(0, CB // L, unroll=8,
                          carry=jnp.zeros((L,), jnp.bool_))
      def viol(i, acc):
        di, v = edge_vec(i)
        cur = plsc.load_gather(agg_col, [di])
        return acc | (v > cur)

      return jnp.any(viol)

    def redo(_):
      scatter_pass()
      return verify_pass()

    scatter_pass()
    lax.while_loop(lambda go: go, redo, verify_pass())

  start(0, 0)

  @pl.loop(0, NSTEP, step=2)
  def _(jj):
    for b in (0, 1):
      j = jj + b

      @pl.when(j + 1 < NSTEP)
      def _():
        start(j + 1, 1 - b)

      wait(j, b)
      process(b)

  pltpu.sync_copy(agg_col, part.at[pl.ds(wid * N, N)])


_sc_aggregate = functools.partial(
    pl.kernel,
    out_type=jax.ShapeDtypeStruct((NW * N,), jnp.float32),
    mesh=plsc.VectorSubcoreMesh(core_axis_name="c", subcore_axis_name="s"),
    compiler_params=pltpu.CompilerParams(needs_layout_passes=False),
    scratch_types=[
        pltpu.VMEM((N,), jnp.float32),
        pltpu.VMEM((N,), jnp.float32),
        pltpu.VMEM((CB,), jnp.int32),
        pltpu.VMEM((CB,), jnp.int32),
        pltpu.VMEM((CB,), jnp.float32),
        pltpu.VMEM((CB,), jnp.float32),
        pltpu.SemaphoreType.DMA((2,)),
    ],
)(_sc_aggregate_body)


_PE = 81920
_GP = -(-E // _PE)
_PN = 3072
_GN = -(-N // _PN)
_BN = 1024


def _prep_body(eaT_ref, sr_ref, dr_ref, w1e_ref, b_ref,
               ec0_ref, ec1_ref, ec2_ref, ec3_ref, sd_ref):
  ecm = _mm(w1e_ref[...], eaT_ref[...]) + b_ref[...]
  ec_refs = (ec0_ref, ec1_ref, ec2_ref, ec3_ref)
  for f in range(FM):
    ec_refs[f][...] = ecm[f]
  sd_ref[...] = sr_ref[...] | lax.shift_left(dr_ref[...], 16)


_prep_call = pl.pallas_call(
    _prep_body,
    grid=(_GP,),
    in_specs=[
        pl.BlockSpec((8, _PE), lambda i: (0, i)),
        pl.BlockSpec((_PE,), lambda i: (i,)),
        pl.BlockSpec((_PE,), lambda i: (i,)),
        pl.BlockSpec((FM, 8), lambda i: (0, 0)),
        pl.BlockSpec((FM, 1), lambda i: (0, 0)),
    ],
    out_specs=[
        pl.BlockSpec((_PE,), lambda i: (i,)),
        pl.BlockSpec((_PE,), lambda i: (i,)),
        pl.BlockSpec((_PE,), lambda i: (i,)),
        pl.BlockSpec((_PE,), lambda i: (i,)),
        pl.BlockSpec((_PE,), lambda i: (i,)),
    ],
    out_shape=[
        jax.ShapeDtypeStruct((E,), jnp.float32),
        jax.ShapeDtypeStruct((E,), jnp.float32),
        jax.ShapeDtypeStruct((E,), jnp.float32),
        jax.ShapeDtypeStruct((E,), jnp.float32),
        jax.ShapeDtypeStruct((E,), jnp.int32),
    ],
)


def _nprep_body(xT_ref, w1_ref, xa0_ref, xa1_ref, xa2_ref, xa3_ref):
  xab = _mm(w1_ref[...], xT_ref[...])
  xa_refs = (xa0_ref, xa1_ref, xa2_ref, xa3_ref)
  for f in range(FM):
    xa_refs[f][...] = xab[f]


_nprep_call = pl.pallas_call(
    _nprep_body,
    grid=(_GN,),
    in_specs=[
        pl.BlockSpec((FX, _PN), lambda i: (0, i)),
        pl.BlockSpec((FM, FX), lambda i: (0, 0)),
    ],
    out_specs=[
        pl.BlockSpec((_PN,), lambda i: (i,)),
        pl.BlockSpec((_PN,), lambda i: (i,)),
        pl.BlockSpec((_PN,), lambda i: (i,)),
        pl.BlockSpec((_PN,), lambda i: (i,)),
    ],
    out_shape=[
        jax.ShapeDtypeStruct((N,), jnp.float32),
        jax.ShapeDtypeStruct((N,), jnp.float32),
        jax.ShapeDtypeStruct((N,), jnp.float32),
        jax.ShapeDtypeStruct((N,), jnp.float32),
    ],
)


def _combine(part_ref, xT_ref, w2ax_ref, w2aa_ref, b2a_ref, w2b_ref, b2b_ref):
  p = part_ref[...]
  aggr = p[:, 0, :]
  for k in range(1, NCHUNK):
    aggr = jnp.maximum(aggr, p[:, k, :])
  xT = xT_ref[...]
  h = _mm(w2ax_ref[...], xT) + _mm(w2aa_ref[...], aggr) + b2a_ref[...]
  h = jnp.maximum(h, 0.0)
  comb = _mm(w2b_ref[...], h) + b2b_ref[...]
  nor = jnp.sqrt(jnp.sum(comb * comb, axis=0, keepdims=True))
  comb = comb / jnp.maximum(1.0, nor)
  return comb, xT[0:8, :]


def _update_body(part_ref, xT_ref, w2ax_ref, w2aa_ref, b2a_ref, w2b_ref,
                 b2b_ref, w1c_ref, w1x_ref, xn_ref,
                 xa0_ref, xa1_ref, xa2_ref, xa3_ref):
  comb, x8 = _combine(part_ref, xT_ref, w2ax_ref, w2aa_ref, b2a_ref,
                      w2b_ref, b2b_ref)
  xn_ref[0:8, :] = comb
  xn_ref[8:16, :] = x8
  xan = _mm(w1c_ref[...], comb) + _mm(w1x_ref[...], x8)
  xa_refs = (xa0_ref, xa1_ref, xa2_ref, xa3_ref)
  for f in range(FM):
    xa_refs[f][...] = xan[f]


_mid_specs = [
    pl.BlockSpec((FM, NCHUNK, _BN), lambda i: (0, 0, i)),
    pl.BlockSpec((FX, _BN), lambda i: (0, i)),
    pl.BlockSpec((FM, FX), lambda i: (0, 0)),
    pl.BlockSpec((FM, FM), lambda i: (0, 0)),
    pl.BlockSpec((FM, 1), lambda i: (0, 0)),
    pl.BlockSpec((8, FM), lambda i: (0, 0)),
    pl.BlockSpec((8, 1), lambda i: (0, 0)),
]

_update_call = pl.pallas_call(
    _update_body,
    grid=(pl.cdiv(N, _BN),),
    in_specs=_mid_specs + [
        pl.BlockSpec((FM, 8), lambda i: (0, 0)),
        pl.BlockSpec((FM, 8), lambda i: (0, 0)),
    ],
    out_specs=[
        pl.BlockSpec((FX, _BN), lambda i: (0, i)),
        pl.BlockSpec((_BN,), lambda i: (i,)),
        pl.BlockSpec((_BN,), lambda i: (i,)),
        pl.BlockSpec((_BN,), lambda i: (i,)),
        pl.BlockSpec((_BN,), lambda i: (i,)),
    ],
    out_shape=[
        jax.ShapeDtypeStruct((FX, N), jnp.float32),
        jax.ShapeDtypeStruct((N,), jnp.float32),
        jax.ShapeDtypeStruct((N,), jnp.float32),
        jax.ShapeDtypeStruct((N,), jnp.float32),
        jax.ShapeDtypeStruct((N,), jnp.float32),
    ],
)


def _final_body(part_ref, xT_ref, w2ax_ref, w2aa_ref, b2a_ref, w2b_ref,
                b2b_ref, xn_ref):
  comb, x8 = _combine(part_ref, xT_ref, w2ax_ref, w2aa_ref, b2a_ref,
                      w2b_ref, b2b_ref)
  xn_ref[0:8, :] = comb
  xn_ref[8:16, :] = x8


_final_call = pl.pallas_call(
    _final_body,
    grid=(pl.cdiv(N, _BN),),
    in_specs=_mid_specs,
    out_specs=pl.BlockSpec((FX, _BN), lambda i: (0, i)),
    out_shape=jax.ShapeDtypeStruct((FX, N), jnp.float32),
)


def kernel(x, edge_index, edge_attr, W1, b1, W2a, b2a, W2b, b2b):
  eaT = jnp.transpose(edge_attr)
  xT = jnp.transpose(x)
  src = edge_index[0]
  dst = edge_index[1]

  w1eT = jnp.transpose(W1[FX:])
  w1xT = jnp.transpose(W1[:FX])

  ec0, ec1, ec2, ec3, sd = _prep_call(eaT, src, dst, w1eT,
                                      b1.reshape(FM, 1))
  xas = _nprep_call(xT, w1xT)

  w2ax = jnp.transpose(W2a[:FX])
  w2aa = jnp.transpose(W2a[FX:])
  w2b = jnp.transpose(W2b)
  b2a_c = b2a.reshape(FM, 1)
  b2b_c = b2b.reshape(8, 1)
  w1c = w1xT[:, 0:8]
  w1x = w1xT[:, 8:16]

  for _ in range(2):
    part = _sc_aggregate(*xas, sd, ec0, ec1, ec2, ec3)
    xT, *xas = _update_call(part.reshape(FM, NCHUNK, N), xT, w2ax, w2aa,
                            b2a_c, w2b, b2b_c, w1c, w1x)

  part = _sc_aggregate(*xas, sd, ec0, ec1, ec2, ec3)
  xnT = _final_call(part.reshape(FM, NCHUNK, N), xT, w2ax, w2aa,
                    b2a_c, w2b, b2b_c)
  return jnp.transpose(xnT)
